# baseline (speedup 1.0000x reference)
import functools

import jax
import jax.numpy as jnp
from jax import lax
from jax.experimental import pallas as pl
from jax.experimental.pallas import tpu as pltpu
from jax.experimental.pallas import tpu_sc as plsc

N_ROWS = 16384
N_COLS = 1000
L = 16


@functools.lru_cache(maxsize=None)
def _make_sc_call():
    info = plsc.get_sparse_core_info()
    nc, ns = info.num_cores, info.num_subcores
    nw = nc * ns
    b_per_w = N_ROWS // nw
    n_chunks = b_per_w // 128
    mesh = plsc.VectorSubcoreMesh(core_axis_name="c", subcore_axis_name="s")

    @functools.partial(
        pl.kernel,
        mesh=mesh,
        out_type=[
            jax.ShapeDtypeStruct((nw, n_chunks, 128), jnp.float32),
            jax.ShapeDtypeStruct((nw, n_chunks, 128), jnp.float32),
        ],
        scratch_types=[
            pltpu.VMEM((n_chunks, 128), jnp.int32),
            pltpu.VMEM((n_chunks, 128), jnp.float32),
            pltpu.VMEM((n_chunks, 128), jnp.float32),
            pltpu.VMEM((n_chunks, 128), jnp.float32),
            pltpu.VMEM((L,), jnp.float32),
            pltpu.SemaphoreType.DMA,
        ],
    )
    def sc_kernel(flat_hbm, tgt_hbm, thr_hbm, om_hbm, w_hbm,
                  idx_v, gath_v, om_v, w_v, thr_v, sem):
        wid = lax.axis_index("s") * nc + lax.axis_index("c")
        base = wid * b_per_w
        pltpu.sync_copy(tgt_hbm.at[wid], idx_v)
        pltpu.sync_copy(thr_hbm, thr_v)
        lane = lax.iota(jnp.int32, L)
        for c in range(n_chunks):
            for k in range(128 // L):
                off = c * 128 + k * L
                t = idx_v[c, pl.ds(k * L, L)]
                idx_v[c, pl.ds(k * L, L)] = (base + off + lane) * N_COLS + t
        copies = [
            pltpu.async_copy(flat_hbm.at[idx_v.at[c]], gath_v.at[c], sem)
            for c in range(n_chunks)
        ]
        for cp in copies:
            cp.wait()
        thr = thr_v[...]
        for c in range(n_chunks):
            for k in range(128 // L):
                w = gath_v[c, pl.ds(k * L, L)] / thr
                w_v[c, pl.ds(k * L, L)] = w
                om_v[c, pl.ds(k * L, L)] = 1.0 - w
        pltpu.sync_copy(om_v, om_hbm.at[wid])
        pltpu.sync_copy(w_v, w_hbm.at[wid])

    return sc_kernel, nw


def kernel(outputs, targets, n_test, threshold):
    del n_test
    sc_call, nw = _make_sc_call()
    flat = outputs.reshape(-1)
    tgt = targets.astype(jnp.int32).reshape(nw, -1, 128)
    thr = jnp.full((L,), threshold, dtype=jnp.float32)
    om, w = sc_call(flat, tgt, thr)
    return om.reshape(-1), w.reshape(-1)


# speedup vs baseline: 1.2870x; 1.2870x over previous
import functools

import jax
import jax.numpy as jnp
from jax import lax
from jax.experimental import pallas as pl
from jax.experimental.pallas import tpu as pltpu
from jax.experimental.pallas import tpu_sc as plsc

N_ROWS = 16384
N_COLS = 1000
L = 16
CH = 32
MAIN_W = 896
BUF_W = 1024


@functools.lru_cache(maxsize=None)
def _make_sc_call():
    info = plsc.get_sparse_core_info()
    nc, ns = info.num_cores, info.num_subcores
    nw = nc * ns
    b_per_w = N_ROWS // nw
    n_chunks = b_per_w // CH
    mesh = plsc.VectorSubcoreMesh(core_axis_name="c", subcore_axis_name="s")

    @functools.partial(
        pl.kernel,
        mesh=mesh,
        out_type=[
            jax.ShapeDtypeStruct((N_ROWS,), jnp.float32),
            jax.ShapeDtypeStruct((N_ROWS,), jnp.float32),
        ],
        scratch_types=[
            pltpu.VMEM((b_per_w,), jnp.int32),
            pltpu.VMEM((CH, BUF_W), jnp.float32),
            pltpu.VMEM((CH, BUF_W), jnp.float32),
            pltpu.VMEM((b_per_w,), jnp.float32),
            pltpu.VMEM((b_per_w,), jnp.float32),
            pltpu.VMEM((L,), jnp.float32),
            pltpu.SemaphoreType.DMA,
            pltpu.SemaphoreType.DMA,
        ],
    )
    def sc_kernel(out2d_hbm, aux_hbm, tgt_hbm, thr_hbm, om_hbm, w_hbm,
                  tv, buf0, buf1, om_v, w_v, thr_v, sem0, sem1):
        wid = lax.axis_index("s") * nc + lax.axis_index("c")
        base = wid * b_per_w
        pltpu.sync_copy(tgt_hbm.at[pl.ds(base, b_per_w)], tv)
        pltpu.sync_copy(thr_hbm, thr_v)
        lane = lax.iota(jnp.int32, L)
        thr = thr_v[...]
        bufs = (buf0, buf1)
        sems = (sem0, sem1)

        def fire(i):
            row = base + i * CH
            return (
                pltpu.async_copy(
                    out2d_hbm.at[pl.ds(row, CH), pl.ds(0, MAIN_W)],
                    bufs[i % 2].at[:, pl.ds(0, MAIN_W)], sems[i % 2]),
                pltpu.async_copy(
                    aux_hbm.at[pl.ds(row, CH), :],
                    bufs[i % 2].at[:, pl.ds(MAIN_W, 128)], sems[i % 2]),
            )

        def extract(i):
            buf = bufs[i % 2]

            @pl.loop(0, CH // L)
            def _rows(k):
                s = i * CH + k * L
                t16 = tv[pl.ds(s, L)]
                acc = jnp.zeros((L,), jnp.float32)
                tb = jnp.where(t16 >= MAIN_W, t16 + 24, t16)
                for q in range(L):
                    cq = tb[q]
                    start = pl.multiple_of((cq >> 4) * L, L)
                    v16 = buf[k * L + q, pl.ds(start, L)]
                    g = v16.at[jnp.full((L,), cq & (L - 1), jnp.int32)].get(
                        mode="promise_in_bounds")
                    acc = jnp.where(lane == q, g, acc)
                w = acc / thr
                w_v[pl.ds(s, L)] = w
                om_v[pl.ds(s, L)] = 1.0 - w

        cps = [fire(0), fire(1)]
        for i in range(n_chunks):
            for cp in cps[i]:
                cp.wait()
            if i + 2 < n_chunks:
                cps.append(fire(i + 2))
            extract(i)

        pltpu.sync_copy(om_v, om_hbm.at[pl.ds(base, b_per_w)])
        pltpu.sync_copy(w_v, w_hbm.at[pl.ds(base, b_per_w)])

    return sc_kernel


def kernel(outputs, targets, n_test, threshold):
    del n_test
    sc_call = _make_sc_call()
    tgt = targets.astype(jnp.int32)
    thr = jnp.full((L,), threshold, dtype=jnp.float32)
    aux = jax.lax.slice(outputs, (0, N_COLS - 128), (N_ROWS, N_COLS))
    om, w = sc_call(outputs, aux, tgt, thr)
    return om, w


# speedup vs baseline: 1.3007x; 1.0107x over previous
import functools

import jax
import jax.numpy as jnp
from jax import lax
from jax.experimental import pallas as pl
from jax.experimental.pallas import tpu as pltpu
from jax.experimental.pallas import tpu_sc as plsc

N_ROWS = 16384
N_COLS = 1000
L = 16
CH = 32
MAIN_W = 896
BUF_W = 1024


@functools.lru_cache(maxsize=None)
def _make_sc_call():
    info = plsc.get_sparse_core_info()
    nc, ns = info.num_cores, info.num_subcores
    nw = nc * ns
    b_per_w = N_ROWS // nw
    n_chunks = b_per_w // CH
    mesh = plsc.VectorSubcoreMesh(core_axis_name="c", subcore_axis_name="s")

    @functools.partial(
        pl.kernel,
        mesh=mesh,
        out_type=[
            jax.ShapeDtypeStruct((N_ROWS,), jnp.float32),
            jax.ShapeDtypeStruct((N_ROWS,), jnp.float32),
        ],
        scratch_types=[
            pltpu.VMEM((b_per_w,), jnp.int32),
            pltpu.VMEM((CH, BUF_W), jnp.float32),
            pltpu.VMEM((CH, BUF_W), jnp.float32),
            pltpu.VMEM((CH, BUF_W), jnp.float32),
            pltpu.VMEM((b_per_w,), jnp.float32),
            pltpu.VMEM((b_per_w,), jnp.float32),
            pltpu.VMEM((L,), jnp.float32),
            pltpu.SemaphoreType.DMA,
            pltpu.SemaphoreType.DMA,
            pltpu.SemaphoreType.DMA,
        ],
    )
    def sc_kernel(out2d_hbm, aux_hbm, tgt_hbm, thr_hbm, om_hbm, w_hbm,
                  tv, buf0, buf1, buf2, om_v, w_v, thr_v, sem0, sem1, sem2):
        wid = lax.axis_index("s") * nc + lax.axis_index("c")
        base = wid * b_per_w
        pltpu.sync_copy(tgt_hbm.at[pl.ds(base, b_per_w)], tv)
        pltpu.sync_copy(thr_hbm, thr_v)
        lane = lax.iota(jnp.int32, L)
        thr = thr_v[...]
        bufs = (buf0, buf1, buf2)
        sems = (sem0, sem1, sem2)
        nb = len(bufs)

        def fire(i):
            row = base + i * CH
            return (
                pltpu.async_copy(
                    out2d_hbm.at[pl.ds(row, CH), pl.ds(0, MAIN_W)],
                    bufs[i % nb].at[:, pl.ds(0, MAIN_W)], sems[i % nb]),
                pltpu.async_copy(
                    aux_hbm.at[pl.ds(row, CH), :],
                    bufs[i % nb].at[:, pl.ds(MAIN_W, 128)], sems[i % nb]),
            )

        def extract(i):
            buf = bufs[i % nb]

            @pl.loop(0, CH // L)
            def _rows(k):
                s = i * CH + k * L
                t16 = tv[pl.ds(s, L)]
                acc = jnp.zeros((L,), jnp.float32)
                tb = jnp.where(t16 >= MAIN_W, t16 + 24, t16)
                for q in range(L):
                    cq = tb[q]
                    start = pl.multiple_of((cq >> 4) * L, L)
                    v16 = buf[k * L + q, pl.ds(start, L)]
                    g = v16.at[jnp.full((L,), cq & (L - 1), jnp.int32)].get(
                        mode="promise_in_bounds")
                    acc = jnp.where(lane == q, g, acc)
                w = acc / thr
                w_v[pl.ds(s, L)] = w
                om_v[pl.ds(s, L)] = 1.0 - w

        cps = [fire(0), fire(1), fire(2)]
        for i in range(n_chunks):
            for cp in cps[i]:
                cp.wait()
            if i + nb < n_chunks:
                cps.append(fire(i + nb))
            extract(i)

        pltpu.sync_copy(om_v, om_hbm.at[pl.ds(base, b_per_w)])
        pltpu.sync_copy(w_v, w_hbm.at[pl.ds(base, b_per_w)])

    return sc_kernel


def kernel(outputs, targets, n_test, threshold):
    del n_test
    sc_call = _make_sc_call()
    tgt = targets.astype(jnp.int32)
    thr = jnp.full((L,), threshold, dtype=jnp.float32)
    aux = jax.lax.slice(outputs, (0, N_COLS - 128), (N_ROWS, N_COLS))
    om, w = sc_call(outputs, aux, tgt, thr)
    return om, w


# speedup vs baseline: 1.3053x; 1.0035x over previous
import functools

import jax
import jax.numpy as jnp
from jax import lax
from jax.experimental import pallas as pl
from jax.experimental.pallas import tpu as pltpu
from jax.experimental.pallas import tpu_sc as plsc

N_ROWS = 16384
N_COLS = 1000
L = 16
CH = 32
MAIN_W = 896
BUF_W = 1024


@functools.lru_cache(maxsize=None)
def _make_sc_call():
    info = plsc.get_sparse_core_info()
    nc, ns = info.num_cores, info.num_subcores
    nw = nc * ns
    b_per_w = N_ROWS // nw
    n_chunks = b_per_w // CH
    mesh = plsc.VectorSubcoreMesh(core_axis_name="c", subcore_axis_name="s")

    @functools.partial(
        pl.kernel,
        mesh=mesh,
        compiler_params=pltpu.CompilerParams(needs_layout_passes=False),
        out_type=[
            jax.ShapeDtypeStruct((N_ROWS,), jnp.float32),
            jax.ShapeDtypeStruct((N_ROWS,), jnp.float32),
        ],
        scratch_types=[
            pltpu.VMEM((b_per_w,), jnp.int32),
            pltpu.VMEM((CH, BUF_W), jnp.float32),
            pltpu.VMEM((CH, BUF_W), jnp.float32),
            pltpu.VMEM((CH, BUF_W), jnp.float32),
            pltpu.VMEM((b_per_w,), jnp.float32),
            pltpu.VMEM((b_per_w,), jnp.float32),
            pltpu.VMEM((L,), jnp.float32),
            pltpu.SemaphoreType.DMA,
            pltpu.SemaphoreType.DMA,
            pltpu.SemaphoreType.DMA,
        ],
    )
    def sc_kernel(out2d_hbm, aux_hbm, tgt_hbm, thr_hbm, om_hbm, w_hbm,
                  tv, buf0, buf1, buf2, om_v, w_v, thr_v, sem0, sem1, sem2):
        wid = lax.axis_index("s") * nc + lax.axis_index("c")
        base = wid * b_per_w
        pltpu.sync_copy(tgt_hbm.at[pl.ds(base, b_per_w)], tv)
        pltpu.sync_copy(thr_hbm, thr_v)
        lane = lax.iota(jnp.int32, L)
        thr = thr_v[...]
        bufs = (buf0, buf1, buf2)
        sems = (sem0, sem1, sem2)
        nb = len(bufs)

        def fire(i):
            row = base + i * CH
            return (
                pltpu.async_copy(
                    out2d_hbm.at[pl.ds(row, CH), pl.ds(0, MAIN_W)],
                    bufs[i % nb].at[:, pl.ds(0, MAIN_W)], sems[i % nb]),
                pltpu.async_copy(
                    aux_hbm.at[pl.ds(row, CH), :],
                    bufs[i % nb].at[:, pl.ds(MAIN_W, 128)], sems[i % nb]),
            )

        def extract(i):
            buf = bufs[i % nb]

            @pl.loop(0, CH // L)
            def _rows(k):
                s = i * CH + k * L
                t16 = tv[pl.ds(s, L)]
                tb = jnp.where(t16 >= MAIN_W, t16 + 24, t16)
                v = plsc.load_gather(buf, [k * L + lane, tb])
                w = v / thr
                w_v[pl.ds(s, L)] = w
                om_v[pl.ds(s, L)] = 1.0 - w

        cps = [fire(0), fire(1), fire(2)]
        for i in range(n_chunks):
            for cp in cps[i]:
                cp.wait()
            if i + nb < n_chunks:
                cps.append(fire(i + nb))
            extract(i)

        pltpu.sync_copy(om_v, om_hbm.at[pl.ds(base, b_per_w)])
        pltpu.sync_copy(w_v, w_hbm.at[pl.ds(base, b_per_w)])

    return sc_kernel


def kernel(outputs, targets, n_test, threshold):
    del n_test
    sc_call = _make_sc_call()
    tgt = targets.astype(jnp.int32)
    thr = jnp.full((L,), threshold, dtype=jnp.float32)
    aux = jax.lax.slice(outputs, (0, N_COLS - 128), (N_ROWS, N_COLS))
    om, w = sc_call(outputs, aux, tgt, thr)
    return om, w


# speedup vs baseline: 1.6238x; 1.2440x over previous
import functools

import jax
import jax.numpy as jnp
from jax import lax
from jax.experimental import pallas as pl
from jax.experimental.pallas import tpu as pltpu
from jax.experimental.pallas import tpu_sc as plsc

N_ROWS = 16384
N_COLS = 1000
L = 16
MAIN_W = 896


@functools.lru_cache(maxsize=None)
def _make_sc_call():
    info = plsc.get_sparse_core_info()
    nc, ns = info.num_cores, info.num_subcores
    nw = nc * ns
    b_per_w = N_ROWS // nw
    mesh = plsc.VectorSubcoreMesh(core_axis_name="c", subcore_axis_name="s")

    @functools.partial(
        pl.kernel,
        mesh=mesh,
        compiler_params=pltpu.CompilerParams(needs_layout_passes=False),
        out_type=[
            jax.ShapeDtypeStruct((N_ROWS,), jnp.float32),
            jax.ShapeDtypeStruct((N_ROWS,), jnp.float32),
        ],
        scratch_types=[
            pltpu.VMEM((b_per_w,), jnp.int32),
            pltpu.VMEM((b_per_w, 128), jnp.float32),
            pltpu.VMEM((b_per_w,), jnp.float32),
            pltpu.VMEM((b_per_w,), jnp.float32),
            pltpu.VMEM((L,), jnp.float32),
            pltpu.SemaphoreType.DMA,
        ],
    )
    def sc_kernel(out2d_hbm, aux_hbm, tgt_hbm, thr_hbm, om_hbm, w_hbm,
                  tv, land, om_v, w_v, thr_v, sem):
        wid = lax.axis_index("s") * nc + lax.axis_index("c")
        base = wid * b_per_w
        pltpu.sync_copy(tgt_hbm.at[pl.ds(base, b_per_w)], tv)
        pltpu.sync_copy(thr_hbm, thr_v)
        lane = lax.iota(jnp.int32, L)
        thr = thr_v[...]

        @pl.loop(0, b_per_w // L)
        def _fire(k):
            s = k * L
            t16 = tv[pl.ds(s, L)]
            for q in range(L):
                cq = t16[q]
                row = base + s + q

                @pl.when(cq < MAIN_W)
                def _main():
                    start = pl.multiple_of((cq >> 7) * 128, 128)
                    pltpu.async_copy(
                        out2d_hbm.at[pl.ds(row, 1), pl.ds(start, 128)],
                        land.at[pl.ds(s + q, 1), :], sem)

                @pl.when(cq >= MAIN_W)
                def _aux():
                    pltpu.async_copy(
                        aux_hbm.at[pl.ds(row, 1), :],
                        land.at[pl.ds(s + q, 1), :], sem)

        @pl.loop(0, b_per_w // L)
        def _drain(k):
            pltpu.make_async_copy(
                out2d_hbm.at[pl.ds(base, L), pl.ds(0, 128)],
                land.at[pl.ds(k * L, L), :], sem).wait()

        @pl.loop(0, b_per_w // L)
        def _extract(k):
            s = k * L
            t16 = tv[pl.ds(s, L)]
            col = jnp.where(t16 >= MAIN_W, t16 - (N_COLS - 128), t16 & 127)
            v = plsc.load_gather(land, [s + lane, col])
            w = v / thr
            w_v[pl.ds(s, L)] = w
            om_v[pl.ds(s, L)] = 1.0 - w

        pltpu.sync_copy(om_v, om_hbm.at[pl.ds(base, b_per_w)])
        pltpu.sync_copy(w_v, w_hbm.at[pl.ds(base, b_per_w)])

    return sc_kernel


def kernel(outputs, targets, n_test, threshold):
    del n_test
    sc_call = _make_sc_call()
    tgt = targets.astype(jnp.int32)
    thr = jnp.full((L,), threshold, dtype=jnp.float32)
    aux = jax.lax.slice(outputs, (0, N_COLS - 128), (N_ROWS, N_COLS))
    om, w = sc_call(outputs, aux, tgt, thr)
    return om, w
